# baseline (speedup 1.0000x reference)
import jax
import jax.numpy as jnp
from jax import lax
from jax.experimental import pallas as pl
from jax.experimental.pallas import tpu as pltpu
from jax.experimental.pallas import tpu_sc as plsc

_VOCAB = 1000000
_EMBED = 64
_BATCH = 16384
_SEQ = 200
_RPB = 64
_NUM_BLOCKS = _BATCH // _RPB
_NC, _NS = 2, 16
_NW = _NC * _NS
_BPW = _NUM_BLOCKS // _NW
_HALF = _BATCH // 2
_HBLK = _NUM_BLOCKS // 2


def _sc_body(x_hbm, tp_hbm, out_hbm, xrow_v, idxe_v, idxo_v, cnte_v, cnto_v,
             acce_v, acco_v, t1_v, sem_g):
    wid = lax.axis_index("s") * _NC + lax.axis_index("c")
    lanes = lax.iota(jnp.int32, 16)
    zi = jnp.zeros((16,), jnp.int32)
    zf = jnp.zeros((16,), jnp.float32)

    pltpu.sync_copy(tp_hbm.at[pl.ds(0, 1)], t1_v)

    def _stage_build(g, slot):
        pltpu.sync_copy(x_hbm.at[pl.ds(g * _RPB, _RPB)], xrow_v.at[slot])

        def _zero(s, c):
            for k in range(4):
                idxe_v[slot, s, pl.ds(16 * k, 16)] = zi
                idxo_v[slot, s, pl.ds(16 * k, 16)] = zi
            return c
        lax.fori_loop(0, _SEQ, _zero, 0)
        for k in range(5):
            cnte_v[slot, pl.ds(16 * k, 16)] = zi
            cnto_v[slot, pl.ds(16 * k, 16)] = zi

        def _build(s, c):
            cols = zi + s
            for k in range(4):
                ln = lanes + (16 * k)
                v = plsc.load_gather(xrow_v.at[slot], [ln, cols])
                u = lax.shift_right_logical(v, 1)
                odd = (v & 1) == 1
                even = jnp.logical_not(odd)
                ce = cnte_v[slot, pl.ds(16 * k, 16)]
                co = cnto_v[slot, pl.ds(16 * k, 16)]
                plsc.store_scatter(idxe_v.at[slot], [ce, ln], u, mask=even)
                plsc.store_scatter(idxo_v.at[slot], [co, ln], u, mask=odd)
                cnte_v[slot, pl.ds(16 * k, 16)] = ce + even.astype(jnp.int32)
                cnto_v[slot, pl.ds(16 * k, 16)] = co + odd.astype(jnp.int32)
            return c
        lax.fori_loop(0, _SEQ, _build, 0)

    def _maxcnt(ref, slot):
        m = ref[slot, pl.ds(0, 16)]
        for k in range(1, 4):
            m = jnp.maximum(m, ref[slot, pl.ds(16 * k, 16)])
        return lax.reduce_max(m, axes=(0,))

    _stage_build(wid * _BPW, 0)

    for t in range(_BPW):
        slot = t % 2
        g = wid * _BPW + t

        def _zacc(i, c):
            for m in range(8):
                acce_v[i, pl.ds(16 * m, 16)] = zf
                acco_v[i, pl.ds(16 * m, 16)] = zf
            return c
        lax.fori_loop(0, _RPB, _zacc, 0)

        max_e = _maxcnt(cnte_v, slot)
        max_o = _maxcnt(cnto_v, slot)

        def _fire_e(k, c):
            pltpu.async_copy(
                tp_hbm.at[idxe_v.at[slot, k]], acce_v, sem_g, add=True)
            return c
        lax.fori_loop(0, max_e, _fire_e, 0)

        def _fire_o(k, c):
            pltpu.async_copy(
                tp_hbm.at[idxo_v.at[slot, k]], acco_v, sem_g, add=True)
            return c
        lax.fori_loop(0, max_o, _fire_o, 0)

        if t + 1 < _BPW:
            _stage_build(g + 1, 1 - slot)

        def _drain(k, c):
            pltpu.make_async_copy(
                tp_hbm.at[idxe_v.at[slot, 0]], acce_v, sem_g).wait()
            return c
        lax.fori_loop(0, max_e + max_o, _drain, 0)

        max_o_f = max_o.astype(jnp.float32)

        def _comb(i, c):
            c16 = cnto_v[slot, pl.ds(i, 16)]
            d_o = max_o_f - c16[0].astype(jnp.float32)
            for m in range(4):
                acce_v[i, pl.ds(16 * m, 16)] = (
                    acce_v[i, pl.ds(16 * m, 16)]
                    + acco_v[i, pl.ds(64 + 16 * m, 16)]
                    - d_o * t1_v[0, pl.ds(64 + 16 * m, 16)])
            return c
        lax.fori_loop(0, _RPB, _comb, 0)

        pltpu.sync_copy(
            acce_v.at[:, pl.ds(0, _EMBED)],
            out_hbm.at[pl.ds((g % _HBLK) * _RPB, _RPB),
                       pl.ds(_EMBED * (g // _HBLK), _EMBED)])


def _sc_pool(x, table_pairs):
    mesh = plsc.VectorSubcoreMesh(core_axis_name="c", subcore_axis_name="s")
    f = pl.kernel(
        _sc_body,
        out_type=jax.ShapeDtypeStruct((_HALF, 2 * _EMBED), jnp.float32),
        mesh=mesh,
        scratch_types=[
            pltpu.VMEM((2, _RPB, _SEQ), jnp.int32),
            pltpu.VMEM((2, _SEQ, _RPB), jnp.int32),
            pltpu.VMEM((2, _SEQ, _RPB), jnp.int32),
            pltpu.VMEM((2, _RPB + 16), jnp.int32),
            pltpu.VMEM((2, _RPB + 16), jnp.int32),
            pltpu.VMEM((_RPB, 2 * _EMBED), jnp.float32),
            pltpu.VMEM((_RPB, 2 * _EMBED), jnp.float32),
            pltpu.VMEM((1, 2 * _EMBED), jnp.float32),
            pltpu.SemaphoreType.DMA,
        ],
        compiler_params=pltpu.CompilerParams(
            use_tc_tiling_on_sc=False, needs_layout_passes=False),
    )
    return f(x, table_pairs)


def _tc_head_body(xa_ref, xb_ref, sp_ref, w1p_ref, b1p_ref, w2p_ref, b2_ref,
                  o_ref):
    cnt_a = jnp.sum((xa_ref[...] != 0).astype(jnp.float32), axis=1,
                    keepdims=True)
    cnt_b = jnp.sum((xb_ref[...] != 0).astype(jnp.float32), axis=1,
                    keepdims=True)
    n = sp_ref.shape[0]
    inv = jnp.concatenate(
        [jnp.broadcast_to(1.0 / jnp.maximum(cnt_a, 1.0), (n, _EMBED)),
         jnp.broadcast_to(1.0 / jnp.maximum(cnt_b, 1.0), (n, _EMBED))],
        axis=1)
    pooled = sp_ref[...] * inv
    h = jnp.dot(pooled, w1p_ref[...], preferred_element_type=jnp.float32)
    h = jnp.maximum(h + b1p_ref[...], 0.0)
    o_ref[...] = (
        jnp.dot(h, w2p_ref[...], preferred_element_type=jnp.float32)
        + b2_ref[...])


def _tc_head(x, sp, w1p, b1p, w2p, b2):
    blk = 1024
    nblk = _HALF // blk
    return pl.pallas_call(
        _tc_head_body,
        grid=(nblk,),
        in_specs=[
            pl.BlockSpec((blk, _SEQ), lambda i: (i, 0)),
            pl.BlockSpec((blk, _SEQ), lambda i: (i + nblk, 0)),
            pl.BlockSpec((blk, 2 * _EMBED), lambda i: (i, 0)),
            pl.BlockSpec((2 * _EMBED, 2 * _EMBED), lambda i: (0, 0)),
            pl.BlockSpec((1, 2 * _EMBED), lambda i: (0, 0)),
            pl.BlockSpec((2 * _EMBED, 2), lambda i: (0, 0)),
            pl.BlockSpec((1, 2), lambda i: (0, 0)),
        ],
        out_specs=pl.BlockSpec((blk, 2), lambda i: (i, 0)),
        out_shape=jax.ShapeDtypeStruct((_HALF, 2), jnp.float32),
    )(x, x, sp, w1p, b1p, w2p, b2)


def kernel(x, table, W1, b1, W2, b2):
    table_pairs = table.reshape(_VOCAB // 2, 2 * _EMBED)
    sp = _sc_pool(x, table_pairs)
    z = jnp.zeros((_EMBED, _EMBED), jnp.float32)
    w1p = jnp.block([[W1.T, z], [z, W1.T]])
    b1p = jnp.concatenate([b1, b1]).reshape(1, 2 * _EMBED)
    zc = jnp.zeros((_EMBED, 1), jnp.float32)
    w2p = jnp.block([[W2.T, zc], [zc, W2.T]])
    b2p = jnp.broadcast_to(b2.reshape(1, 1), (1, 2))
    out2 = _tc_head(x, sp, w1p, b1p, w2p, b2p)
    return jnp.concatenate([out2[:, :1], out2[:, 1:]], axis=0)
